# Initial kernel scaffold; baseline (speedup 1.0000x reference)
import jax
import jax.numpy as jnp
from jax.experimental import pallas as pl


def kernel(features, batch_nodes, s1_neighs, s2_neighs, W1, b1, W2, b2):
    raise NotImplementedError("write your pallas kernel here")


# speedup vs baseline: 9.3950x; 9.3950x over previous
import functools

import jax
import jax.numpy as jnp
from jax import lax
from jax.experimental import pallas as pl
from jax.experimental.pallas import tpu as pltpu
from jax.experimental.pallas import tpu_sc as plsc

N_NODES = 100000
D_FEAT = 128
DIMS = 32
B = 2048
S = 16

NW = 32
SEGS = B * S
SEG_PER_W = SEGS // NW
CHUNK_SEGS = 8
CHUNKS = SEG_PER_W // CHUNK_SEGS
ROWS_PER_CHUNK = CHUNK_SEGS * S


def _t_body(f_ref, w_ref, b_ref, o_ref):
    x = jnp.dot(f_ref[...], w_ref[...], preferred_element_type=jnp.float32)
    o_ref[...] = jnp.maximum(x + b_ref[0:1, :], 0.0)


def _compute_t(features, W1, b1):
    blk = 2000
    b1b = jnp.broadcast_to(b1.reshape(1, DIMS), (8, DIMS))
    return pl.pallas_call(
        _t_body,
        grid=(N_NODES // blk,),
        in_specs=[
            pl.BlockSpec((blk, D_FEAT), lambda i: (i, 0)),
            pl.BlockSpec((D_FEAT, DIMS), lambda i: (0, 0)),
            pl.BlockSpec((8, DIMS), lambda i: (0, 0)),
        ],
        out_specs=pl.BlockSpec((blk, DIMS), lambda i: (i, 0)),
        out_shape=jax.ShapeDtypeStruct((N_NODES, DIMS), jnp.float32),
    )(features, W1, b1b)


def _sc_body(t_hbm, s2i_hbm, s1i_hbm, bi_hbm, sum2_o, ts1_o, tb_o,
             idx2, idx1, idxb, buf_a, buf_b, buf_c, outb, sem_a, sem_b):
    wid = lax.axis_index("s") * 2 + lax.axis_index("c")

    pltpu.sync_copy(s2i_hbm.at[pl.ds(wid * CHUNKS, CHUNKS)], idx2)
    pltpu.sync_copy(s1i_hbm.at[pl.ds(wid * 8, 8)], idx1)
    pltpu.sync_copy(bi_hbm.at[wid], idxb)

    def process(buf, c):
        for k in range(CHUNK_SEGS):
            r = k * S
            a0 = buf[r, pl.ds(0, 16)]
            a1 = buf[r, pl.ds(16, 16)]
            for j in range(1, S):
                a0 = a0 + buf[r + j, pl.ds(0, 16)]
                a1 = a1 + buf[r + j, pl.ds(16, 16)]
            outb[c * CHUNK_SEGS + k, pl.ds(0, 16)] = a0
            outb[c * CHUNK_SEGS + k, pl.ds(16, 16)] = a1

    pltpu.async_copy(t_hbm.at[idx2.at[0]], buf_a, sem_a)

    def body(c2, carry):
        c = c2 * 2
        pltpu.async_copy(t_hbm.at[idx2.at[c + 1]], buf_b, sem_b)
        pltpu.make_async_copy(t_hbm.at[idx2.at[c]], buf_a, sem_a).wait()
        process(buf_a, c)

        @pl.when(c2 < CHUNKS // 2 - 1)
        def _():
            pltpu.async_copy(t_hbm.at[idx2.at[c + 2]], buf_a, sem_a)

        pltpu.make_async_copy(t_hbm.at[idx2.at[c + 1]], buf_b, sem_b).wait()
        process(buf_b, c + 1)
        return carry

    lax.fori_loop(0, CHUNKS // 2, body, 0)
    pltpu.sync_copy(outb, sum2_o.at[pl.ds(wid * SEG_PER_W, SEG_PER_W)])

    for c in range(8):
        pltpu.async_copy(t_hbm.at[idx1.at[c]], buf_a, sem_a).wait()
        pltpu.sync_copy(buf_a, ts1_o.at[pl.ds(wid * SEG_PER_W + c * 128, 128)])

    pltpu.async_copy(t_hbm.at[idxb], buf_c, sem_a).wait()
    pltpu.sync_copy(buf_c, tb_o.at[pl.ds(wid * (B // NW), B // NW)])


_sc_gather = functools.partial(
    pl.kernel,
    out_type=(
        jax.ShapeDtypeStruct((SEGS, DIMS), jnp.float32),
        jax.ShapeDtypeStruct((SEGS, DIMS), jnp.float32),
        jax.ShapeDtypeStruct((B, DIMS), jnp.float32),
    ),
    mesh=plsc.VectorSubcoreMesh(core_axis_name="c", subcore_axis_name="s"),
    compiler_params=pltpu.CompilerParams(use_tc_tiling_on_sc=False),
    scratch_types=[
        pltpu.VMEM((CHUNKS, ROWS_PER_CHUNK), jnp.int32),
        pltpu.VMEM((8, 128), jnp.int32),
        pltpu.VMEM((B // NW,), jnp.int32),
        pltpu.VMEM((ROWS_PER_CHUNK, DIMS), jnp.float32),
        pltpu.VMEM((ROWS_PER_CHUNK, DIMS), jnp.float32),
        pltpu.VMEM((B // NW, DIMS), jnp.float32),
        pltpu.VMEM((SEG_PER_W, DIMS), jnp.float32),
        pltpu.SemaphoreType.DMA,
        pltpu.SemaphoreType.DMA,
    ],
)(_sc_body)


def _s3_body(s2_ref, t1_ref, tb_ref, w2_ref, b2_ref, o_ref):
    w2 = w2_ref[...]
    b2v = b2_ref[0:1, :]
    acc_l = jnp.zeros(tb_ref.shape, jnp.float32)
    acc_s = jnp.zeros(tb_ref.shape, jnp.float32)
    for s in range(S):
        t1 = t1_ref[s]
        an1 = (s2_ref[s] + t1) * (1.0 / 17.0)
        h = jnp.maximum(jnp.dot(an1, w2, preferred_element_type=jnp.float32) + b2v, 0.0)
        acc_l = acc_l + h
        acc_s = acc_s + t1
    an0 = (acc_s + tb_ref[...]) * (1.0 / 17.0)
    h0 = jnp.maximum(jnp.dot(an0, w2, preferred_element_type=jnp.float32) + b2v, 0.0)
    o_ref[...] = (acc_l + h0) * (1.0 / 17.0)


def _stage3(sum2, ts1, tb, W2, b2):
    blk = 256
    b2b = jnp.broadcast_to(b2.reshape(1, DIMS), (8, DIMS))
    return pl.pallas_call(
        _s3_body,
        grid=(B // blk,),
        in_specs=[
            pl.BlockSpec((S, blk, DIMS), lambda i: (0, i, 0)),
            pl.BlockSpec((S, blk, DIMS), lambda i: (0, i, 0)),
            pl.BlockSpec((blk, DIMS), lambda i: (i, 0)),
            pl.BlockSpec((DIMS, DIMS), lambda i: (0, 0)),
            pl.BlockSpec((8, DIMS), lambda i: (0, 0)),
        ],
        out_specs=pl.BlockSpec((blk, DIMS), lambda i: (i, 0)),
        out_shape=jax.ShapeDtypeStruct((B, DIMS), jnp.float32),
    )(sum2.reshape(S, B, DIMS), ts1.reshape(S, B, DIMS), tb, W2, b2b)


def kernel(features, batch_nodes, s1_neighs, s2_neighs, W1, b1, W2, b2):
    T = _compute_t(features, W1, b1)

    s2t = jnp.transpose(s2_neighs, (2, 0, 1)).astype(jnp.int32)
    s2i = s2t.reshape(NW * CHUNKS, ROWS_PER_CHUNK)
    s1i = jnp.transpose(s1_neighs, (1, 0)).astype(jnp.int32).reshape(NW * 8, 128)
    bi = batch_nodes.astype(jnp.int32).reshape(NW, B // NW)

    sum2, ts1, tb = _sc_gather(T, s2i, s1i, bi)
    return _stage3(sum2, ts1, tb, W2, b2)


# speedup vs baseline: 9.4400x; 1.0048x over previous
import functools

import jax
import jax.numpy as jnp
from jax import lax
from jax.experimental import pallas as pl
from jax.experimental.pallas import tpu as pltpu
from jax.experimental.pallas import tpu_sc as plsc

N_NODES = 100000
D_FEAT = 128
DIMS = 32
B = 2048
S = 16

NW = 32
SEGS = B * S
SEG_PER_W = SEGS // NW
CHUNK_SEGS = 8
CHUNKS = SEG_PER_W // CHUNK_SEGS
ROWS_PER_CHUNK = CHUNK_SEGS * S


def _t_body(f_ref, w_ref, b_ref, o_ref):
    x = jnp.dot(f_ref[...], w_ref[...], preferred_element_type=jnp.float32)
    o_ref[...] = jnp.maximum(x + b_ref[0:1, :], 0.0).astype(jnp.bfloat16)


def _compute_t(features, W1, b1):
    blk = 5000
    b1b = jnp.broadcast_to(b1.reshape(1, DIMS), (8, DIMS))
    return pl.pallas_call(
        _t_body,
        grid=(N_NODES // blk,),
        in_specs=[
            pl.BlockSpec((blk, D_FEAT), lambda i: (i, 0)),
            pl.BlockSpec((D_FEAT, DIMS), lambda i: (0, 0)),
            pl.BlockSpec((8, DIMS), lambda i: (0, 0)),
        ],
        out_specs=pl.BlockSpec((blk, DIMS), lambda i: (i, 0)),
        out_shape=jax.ShapeDtypeStruct((N_NODES, DIMS), jnp.bfloat16),
    )(features, W1, b1b)


def _sc_body(t_hbm, s2i_hbm, s1i_hbm, bi_hbm, sum2_o, ts1_o, tb_o,
             idx2, idx1, idxb, buf_a, buf_b, buf_c, outb, outc, sem_a, sem_b):
    wid = lax.axis_index("s") * 2 + lax.axis_index("c")

    pltpu.sync_copy(s2i_hbm.at[pl.ds(wid * CHUNKS, CHUNKS)], idx2)
    pltpu.sync_copy(s1i_hbm.at[pl.ds(wid * 8, 8)], idx1)
    pltpu.sync_copy(bi_hbm.at[wid], idxb)

    def unpk(row):
        return plsc.unpack(row, format=plsc.PackFormat.INTERLEAVED,
                           preferred_element_type=jnp.float32)

    def process(buf, c):
        for k in range(CHUNK_SEGS):
            r = k * S
            a0, a1 = unpk(buf[r])
            for j in range(1, S):
                u0, u1 = unpk(buf[r + j])
                a0 = a0 + u0
                a1 = a1 + u1
            outb[c * CHUNK_SEGS + k, pl.ds(0, 16)] = a0
            outb[c * CHUNK_SEGS + k, pl.ds(16, 16)] = a1

    pltpu.async_copy(t_hbm.at[idx2.at[0]], buf_a, sem_a)

    def body(c2, carry):
        c = c2 * 2
        pltpu.async_copy(t_hbm.at[idx2.at[c + 1]], buf_b, sem_b)
        pltpu.make_async_copy(t_hbm.at[idx2.at[c]], buf_a, sem_a).wait()
        process(buf_a, c)

        @pl.when(c2 < CHUNKS // 2 - 1)
        def _():
            pltpu.async_copy(t_hbm.at[idx2.at[c + 2]], buf_a, sem_a)

        pltpu.make_async_copy(t_hbm.at[idx2.at[c + 1]], buf_b, sem_b).wait()
        process(buf_b, c + 1)
        return carry

    lax.fori_loop(0, CHUNKS // 2, body, 0)
    pltpu.sync_copy(outb, sum2_o.at[pl.ds(wid * SEG_PER_W, SEG_PER_W)])

    for c in range(8):
        pltpu.async_copy(t_hbm.at[idx1.at[c]], buf_a, sem_a).wait()
        for r in range(128):
            u0, u1 = unpk(buf_a[r])
            outc[r, pl.ds(0, 16)] = u0
            outc[r, pl.ds(16, 16)] = u1
        pltpu.sync_copy(outc, ts1_o.at[pl.ds(wid * SEG_PER_W + c * 128, 128)])

    pltpu.async_copy(t_hbm.at[idxb], buf_c, sem_a).wait()
    for r in range(B // NW):
        u0, u1 = unpk(buf_c[r])
        outc[r, pl.ds(0, 16)] = u0
        outc[r, pl.ds(16, 16)] = u1
    pltpu.sync_copy(outc.at[pl.ds(0, B // NW)], tb_o.at[pl.ds(wid * (B // NW), B // NW)])


_sc_gather = functools.partial(
    pl.kernel,
    out_type=(
        jax.ShapeDtypeStruct((SEGS, DIMS), jnp.float32),
        jax.ShapeDtypeStruct((SEGS, DIMS), jnp.float32),
        jax.ShapeDtypeStruct((B, DIMS), jnp.float32),
    ),
    mesh=plsc.VectorSubcoreMesh(core_axis_name="c", subcore_axis_name="s"),
    compiler_params=pltpu.CompilerParams(use_tc_tiling_on_sc=False,
                                         needs_layout_passes=False),
    scratch_types=[
        pltpu.VMEM((CHUNKS, ROWS_PER_CHUNK), jnp.int32),
        pltpu.VMEM((8, 128), jnp.int32),
        pltpu.VMEM((B // NW,), jnp.int32),
        pltpu.VMEM((ROWS_PER_CHUNK, DIMS), jnp.bfloat16),
        pltpu.VMEM((ROWS_PER_CHUNK, DIMS), jnp.bfloat16),
        pltpu.VMEM((B // NW, DIMS), jnp.bfloat16),
        pltpu.VMEM((SEG_PER_W, DIMS), jnp.float32),
        pltpu.VMEM((ROWS_PER_CHUNK, DIMS), jnp.float32),
        pltpu.SemaphoreType.DMA,
        pltpu.SemaphoreType.DMA,
    ],
)(_sc_body)


def _s3_body(s2_ref, t1_ref, tb_ref, w2_ref, b2_ref, o_ref):
    w2 = w2_ref[...]
    b2v = b2_ref[0:1, :]
    acc_l = jnp.zeros(tb_ref.shape, jnp.float32)
    acc_s = jnp.zeros(tb_ref.shape, jnp.float32)
    for s in range(S):
        t1 = t1_ref[s]
        an1 = (s2_ref[s] + t1) * (1.0 / 17.0)
        h = jnp.maximum(jnp.dot(an1, w2, preferred_element_type=jnp.float32) + b2v, 0.0)
        acc_l = acc_l + h
        acc_s = acc_s + t1
    an0 = (acc_s + tb_ref[...]) * (1.0 / 17.0)
    h0 = jnp.maximum(jnp.dot(an0, w2, preferred_element_type=jnp.float32) + b2v, 0.0)
    o_ref[...] = (acc_l + h0) * (1.0 / 17.0)


def _stage3(sum2, ts1, tb, W2, b2):
    blk = 256
    b2b = jnp.broadcast_to(b2.reshape(1, DIMS), (8, DIMS))
    return pl.pallas_call(
        _s3_body,
        grid=(B // blk,),
        in_specs=[
            pl.BlockSpec((S, blk, DIMS), lambda i: (0, i, 0)),
            pl.BlockSpec((S, blk, DIMS), lambda i: (0, i, 0)),
            pl.BlockSpec((blk, DIMS), lambda i: (i, 0)),
            pl.BlockSpec((DIMS, DIMS), lambda i: (0, 0)),
            pl.BlockSpec((8, DIMS), lambda i: (0, 0)),
        ],
        out_specs=pl.BlockSpec((blk, DIMS), lambda i: (i, 0)),
        out_shape=jax.ShapeDtypeStruct((B, DIMS), jnp.float32),
    )(sum2.reshape(S, B, DIMS), ts1.reshape(S, B, DIMS), tb, W2, b2b)


_QPERM = [c for i in range(16) for c in (i, 16 + i)]


def kernel(features, batch_nodes, s1_neighs, s2_neighs, W1, b1, W2, b2):
    qp = jnp.array(_QPERM, dtype=jnp.int32)
    T = _compute_t(features, W1[:, qp], b1[qp])

    s2t = jnp.transpose(s2_neighs, (2, 0, 1)).astype(jnp.int32)
    s2i = s2t.reshape(NW * CHUNKS, ROWS_PER_CHUNK)
    s1i = jnp.transpose(s1_neighs, (1, 0)).astype(jnp.int32).reshape(NW * 8, 128)
    bi = batch_nodes.astype(jnp.int32).reshape(NW, B // NW)

    sum2, ts1, tb = _sc_gather(T, s2i, s1i, bi)
    return _stage3(sum2, ts1, tb, W2, b2)
